# baseline (speedup 1.0000x reference)
import functools

import jax
import jax.numpy as jnp
from jax import lax
from jax.experimental import pallas as pl
from jax.experimental.pallas import tpu as pltpu
from jax.experimental.pallas import tpu_sc as plsc

_B, _S, _H, _L = 16, 64, 1024, 2048
_LANES = 16
_NW = 32
_RPW = (_B * _L) // _NW
_CH = 32
_NCH = _RPW // _CH
_NBUF = 3
_ZROW = _B * _S


def _body(table_hbm, wns_hbm, dl_hbm, out_hbm,
          wns_v, dl_v, ends_v, idx_v, bufs, *sems):
    nc = 2
    wid = lax.axis_index("s") * nc + lax.axis_index("c")
    b = wid // 2
    half = wid % 2
    out_base = wid * _RPW
    tpos0 = half * _RPW

    pltpu.sync_copy(wns_hbm.at[b], wns_v)
    pltpu.sync_copy(dl_hbm, dl_v)

    zcarry = jnp.int32(0)
    carry = jnp.int32(0)
    for c in range(_S // _LANES):
        wv = wns_v[pl.ds(c * _LANES, _LANES)]
        cz = plsc.cumsum((wv == 0).astype(jnp.int32)) + zcarry
        eff = jnp.where(cz == 0, wv, 0)
        ce = plsc.cumsum(eff) + carry
        ends_v[pl.ds(c * _LANES, _LANES)] = ce
        zcarry = jnp.max(cz)
        carry = jnp.max(ce)

    dlv = dl_v[...]
    lane = lax.iota(jnp.int32, _LANES)
    e_last = plsc.load_gather(ends_v, [jnp.full((_LANES,), _S - 1, jnp.int32)])

    def idx_group(i, acc):
        t = tpos0 + i * _LANES + lane
        sid = jnp.zeros((_LANES,), jnp.int32)
        for step in (32, 16, 8, 4, 2, 1):
            probe = sid + (step - 1)
            e = plsc.load_gather(ends_v, [probe])
            sid = jnp.where(e <= t, sid + step, sid)
        valid = (e_last > t) & (t < dlv)
        idx = jnp.where(valid, b * _S + sid, _ZROW)
        idx_v[pl.ds(i * _LANES, _LANES)] = idx
        return acc

    lax.fori_loop(0, _RPW // _LANES, idx_group, 0)

    gsems = sems[:_NBUF]
    osems = sems[_NBUF:]

    def g_copy(c):
        slot = c % _NBUF
        return pltpu.make_async_copy(
            table_hbm.at[idx_v.at[pl.ds(c * _CH, _CH)]], bufs.at[slot],
            gsems[slot])

    def o_copy(c):
        slot = c % _NBUF
        return pltpu.make_async_copy(
            bufs.at[slot], out_hbm.at[pl.ds(out_base + c * _CH, _CH)],
            osems[slot])

    pf = _NBUF - 1
    for c in range(min(pf, _NCH)):
        g_copy(c).start()
    for c in range(_NCH):
        nxt = c + pf
        if nxt < _NCH:
            if nxt - _NBUF >= 0:
                o_copy(nxt - _NBUF).wait()
            g_copy(nxt).start()
        g_copy(c).wait()
        o_copy(c).start()
    for c in range(max(0, _NCH - _NBUF), _NCH):
        o_copy(c).wait()


def kernel(doc_s, doc_len, wns):
    table = jnp.concatenate(
        [doc_s.reshape(_B * _S, _H),
         jnp.zeros((8, _H), doc_s.dtype)], axis=0)
    wns32 = wns.astype(jnp.int32)
    dl = jnp.broadcast_to(jnp.asarray(doc_len, jnp.int32), (_LANES,))

    run = functools.partial(
        pl.kernel,
        mesh=plsc.VectorSubcoreMesh(core_axis_name="c", subcore_axis_name="s"),
        compiler_params=pltpu.CompilerParams(needs_layout_passes=False),
        out_type=jax.ShapeDtypeStruct((_B * _L, _H), jnp.float32),
        scratch_types=[
            pltpu.VMEM((_S,), jnp.int32),
            pltpu.VMEM((_LANES,), jnp.int32),
            pltpu.VMEM((_S,), jnp.int32),
            pltpu.VMEM((_RPW,), jnp.int32),
            pltpu.VMEM((_NBUF, _CH, _H), jnp.float32),
        ] + [pltpu.SemaphoreType.DMA] * (2 * _NBUF),
    )(_body)

    out = run(table, wns32, dl)
    return out.reshape(_B, _L, _H)


# speedup vs baseline: 18.6779x; 18.6779x over previous
import functools

import jax
import jax.numpy as jnp
from jax import lax
from jax.experimental import pallas as pl
from jax.experimental.pallas import tpu as pltpu
from jax.experimental.pallas import tpu_sc as plsc

_B, _S, _H, _L = 16, 64, 1024, 2048
_LANES = 16
_NW = 32
_RPW = (_B * _L) // _NW
_ZROW = _S


def _body(doc_hbm, wns_hbm, dl_hbm, out_hbm,
          table_v, wns_v, dl_v, ends_v, idx_v, sem):
    nc = 2
    wid = lax.axis_index("s") * nc + lax.axis_index("c")
    b = wid // 2
    half = wid % 2
    out_base = wid * _RPW
    tpos0 = half * _RPW

    pltpu.sync_copy(wns_hbm.at[b], wns_v)
    pltpu.sync_copy(dl_hbm, dl_v)
    pltpu.sync_copy(doc_hbm.at[b], table_v.at[pl.ds(0, _S)])
    fz = jnp.zeros((_LANES,), jnp.float32)
    for j in range(_H // _LANES):
        table_v[_ZROW, pl.ds(j * _LANES, _LANES)] = fz

    zcarry = jnp.int32(0)
    carry = jnp.int32(0)
    for c in range(_S // _LANES):
        wv = wns_v[pl.ds(c * _LANES, _LANES)]
        cz = plsc.cumsum((wv == 0).astype(jnp.int32)) + zcarry
        eff = jnp.where(cz == 0, wv, 0)
        ce = plsc.cumsum(eff) + carry
        ends_v[pl.ds(c * _LANES, _LANES)] = ce
        zcarry = jnp.max(cz)
        carry = jnp.max(ce)

    dlv = dl_v[...]
    lane = lax.iota(jnp.int32, _LANES)
    e_last = plsc.load_gather(ends_v, [jnp.full((_LANES,), _S - 1, jnp.int32)])

    def idx_group(i, acc):
        t = tpos0 + i * _LANES + lane
        sid = jnp.zeros((_LANES,), jnp.int32)
        for step in (32, 16, 8, 4, 2, 1):
            probe = sid + (step - 1)
            e = plsc.load_gather(ends_v, [probe])
            sid = jnp.where(e <= t, sid + step, sid)
        valid = (e_last > t) & (t < dlv)
        idx_v[pl.ds(i * _LANES, _LANES)] = jnp.where(valid, sid, _ZROW)
        return acc

    lax.fori_loop(0, _RPW // _LANES, idx_group, 0)

    def issue(r, acc):
        srow = jnp.max(plsc.load_gather(idx_v, [jnp.full((_LANES,), r, jnp.int32)]))
        pltpu.make_async_copy(
            table_v.at[pl.ds(srow, 1)],
            out_hbm.at[pl.ds(out_base + r, 1)],
            sem).start()
        return acc

    lax.fori_loop(0, _RPW, issue, 0)

    def drain(r, acc):
        pltpu.make_async_copy(
            table_v.at[pl.ds(0, 1)],
            out_hbm.at[pl.ds(out_base, 1)],
            sem).wait()
        return acc

    lax.fori_loop(0, _RPW, drain, 0)


def kernel(doc_s, doc_len, wns):
    wns32 = wns.astype(jnp.int32)
    dl = jnp.broadcast_to(jnp.asarray(doc_len, jnp.int32), (_LANES,))

    run = functools.partial(
        pl.kernel,
        mesh=plsc.VectorSubcoreMesh(core_axis_name="c", subcore_axis_name="s"),
        compiler_params=pltpu.CompilerParams(needs_layout_passes=False),
        out_type=jax.ShapeDtypeStruct((_B * _L, _H), jnp.float32),
        scratch_types=[
            pltpu.VMEM((_S + 1, _H), jnp.float32),
            pltpu.VMEM((_S,), jnp.int32),
            pltpu.VMEM((_LANES,), jnp.int32),
            pltpu.VMEM((_S,), jnp.int32),
            pltpu.VMEM((_RPW,), jnp.int32),
            pltpu.SemaphoreType.DMA,
        ],
    )(_body)

    out = run(doc_s, wns32, dl)
    return out.reshape(_B, _L, _H)


# speedup vs baseline: 19.7239x; 1.0560x over previous
import functools

import jax
import jax.numpy as jnp
from jax import lax
from jax.experimental import pallas as pl
from jax.experimental.pallas import tpu as pltpu
from jax.experimental.pallas import tpu_sc as plsc

_B, _S, _H, _L = 16, 64, 1024, 2048
_LANES = 16
_NW = 32
_RPW = (_B * _L) // _NW
_ZROW = _S
_ZN = 16


def _body(doc_hbm, wns_hbm, dl_hbm, out_hbm,
          table_v, wns_v, dl_v, ends_v, idx_v, sem):
    nc = 2
    wid = lax.axis_index("s") * nc + lax.axis_index("c")
    b = wid // 2
    half = wid % 2
    out_base = wid * _RPW
    tpos0 = half * _RPW

    pltpu.sync_copy(wns_hbm.at[b], wns_v)
    pltpu.sync_copy(dl_hbm, dl_v)
    pltpu.sync_copy(doc_hbm.at[b], table_v.at[pl.ds(0, _S)])
    fz = jnp.zeros((_LANES,), jnp.float32)

    def zrow(i, acc):
        for j in range(_H // _LANES):
            table_v[_ZROW + i, pl.ds(j * _LANES, _LANES)] = fz
        return acc

    lax.fori_loop(0, _ZN, zrow, 0)

    zcarry = jnp.int32(0)
    carry = jnp.int32(0)
    for c in range(_S // _LANES):
        wv = wns_v[pl.ds(c * _LANES, _LANES)]
        cz = plsc.cumsum((wv == 0).astype(jnp.int32)) + zcarry
        eff = jnp.where(cz == 0, wv, 0)
        ce = plsc.cumsum(eff) + carry
        ends_v[pl.ds(c * _LANES, _LANES)] = ce
        zcarry = jnp.max(cz)
        carry = jnp.max(ce)

    dlv = dl_v[...]
    lane = lax.iota(jnp.int32, _LANES)
    e_last = plsc.load_gather(ends_v, [jnp.full((_LANES,), _S - 1, jnp.int32)])

    def idx_group(i, acc):
        t = tpos0 + i * _LANES + lane
        sid = jnp.zeros((_LANES,), jnp.int32)
        for step in (32, 16, 8, 4, 2, 1):
            probe = sid + (step - 1)
            e = plsc.load_gather(ends_v, [probe])
            sid = jnp.where(e <= t, sid + step, sid)
        valid = (e_last > t) & (t < dlv)
        idx_v[pl.ds(i * _LANES, _LANES)] = jnp.where(valid, sid, _ZROW)
        return acc

    lax.fori_loop(0, _RPW // _LANES, idx_group, 0)

    total = carry
    dl_s = jnp.max(dlv)
    bound = jnp.clip(jnp.minimum(total, dl_s) - tpos0, 0, _RPW)
    bound_up = ((bound + _ZN - 1) // _ZN) * _ZN

    def issue(r, acc):
        srow = jnp.max(plsc.load_gather(idx_v, [jnp.full((_LANES,), r, jnp.int32)]))
        pltpu.make_async_copy(
            table_v.at[pl.ds(srow, 1)],
            out_hbm.at[pl.ds(out_base + r, 1)],
            sem).start()
        return acc

    lax.fori_loop(0, bound_up, issue, 0)

    def issue_zero(g, acc):
        pltpu.make_async_copy(
            table_v.at[pl.ds(_ZROW, _ZN)],
            out_hbm.at[pl.ds(out_base + g * _ZN, _ZN)],
            sem).start()
        return acc

    lax.fori_loop(bound_up // _ZN, _RPW // _ZN, issue_zero, 0)

    def drain(g, acc):
        pltpu.make_async_copy(
            table_v.at[pl.ds(_ZROW, _ZN)],
            out_hbm.at[pl.ds(out_base, _ZN)],
            sem).wait()
        return acc

    lax.fori_loop(0, _RPW // _ZN, drain, 0)


def kernel(doc_s, doc_len, wns):
    wns32 = wns.astype(jnp.int32)
    dl = jnp.broadcast_to(jnp.asarray(doc_len, jnp.int32), (_LANES,))

    run = functools.partial(
        pl.kernel,
        mesh=plsc.VectorSubcoreMesh(core_axis_name="c", subcore_axis_name="s"),
        compiler_params=pltpu.CompilerParams(needs_layout_passes=False),
        out_type=jax.ShapeDtypeStruct((_B * _L, _H), jnp.float32),
        scratch_types=[
            pltpu.VMEM((_S + _ZN, _H), jnp.float32),
            pltpu.VMEM((_S,), jnp.int32),
            pltpu.VMEM((_LANES,), jnp.int32),
            pltpu.VMEM((_S,), jnp.int32),
            pltpu.VMEM((_RPW,), jnp.int32),
            pltpu.SemaphoreType.DMA,
        ],
    )(_body)

    out = run(doc_s, wns32, dl)
    return out.reshape(_B, _L, _H)


# speedup vs baseline: 20.5614x; 1.0425x over previous
import functools

import jax
import jax.numpy as jnp
from jax import lax
from jax.experimental import pallas as pl
from jax.experimental.pallas import tpu as pltpu
from jax.experimental.pallas import tpu_sc as plsc

_B, _S, _H, _L = 16, 64, 1024, 2048
_LANES = 16
_NW = 32
_RPW = (_B * _L) // _NW
_ZROW = _S
_ZN = 16


def _body(doc_hbm, wns_hbm, dl_hbm, out_hbm,
          table_v, wns_v, dl_v, ends_v, idx_v, sem, tsem):
    nc = 2
    wid = lax.axis_index("s") * nc + lax.axis_index("c")
    b = wid // 2
    half = wid % 2
    out_base = wid * _RPW
    tpos0 = half * _RPW

    pltpu.sync_copy(wns_hbm.at[b], wns_v)
    pltpu.sync_copy(dl_hbm, dl_v)
    table_cp = pltpu.make_async_copy(
        doc_hbm.at[b], table_v.at[pl.ds(0, _S)], tsem)
    table_cp.start()
    fz = jnp.zeros((_LANES,), jnp.float32)

    def zrow(i, acc):
        for j in range(_H // _LANES):
            table_v[_ZROW + i, pl.ds(j * _LANES, _LANES)] = fz
        return acc

    lax.fori_loop(0, _ZN, zrow, 0)

    zcarry = jnp.int32(0)
    carry = jnp.int32(0)
    for c in range(_S // _LANES):
        wv = wns_v[pl.ds(c * _LANES, _LANES)]
        cz = plsc.cumsum((wv == 0).astype(jnp.int32)) + zcarry
        eff = jnp.where(cz == 0, wv, 0)
        ce = plsc.cumsum(eff) + carry
        ends_v[pl.ds(c * _LANES, _LANES)] = ce
        zcarry = jnp.max(cz)
        carry = jnp.max(ce)

    dlv = dl_v[...]
    lane = lax.iota(jnp.int32, _LANES)
    e_last = plsc.load_gather(ends_v, [jnp.full((_LANES,), _S - 1, jnp.int32)])

    def idx_group(i, acc):
        t = tpos0 + i * _LANES + lane
        sid = jnp.zeros((_LANES,), jnp.int32)
        for step in (32, 16, 8, 4, 2, 1):
            probe = sid + (step - 1)
            e = plsc.load_gather(ends_v, [probe])
            sid = jnp.where(e <= t, sid + step, sid)
        valid = (e_last > t) & (t < dlv)
        idx_v[pl.ds(i * _LANES, _LANES)] = jnp.where(valid, sid, _ZROW)
        return acc

    lax.fori_loop(0, _RPW // _LANES, idx_group, 0)

    total = carry
    dl_s = jnp.max(dlv)
    bound = jnp.clip(jnp.minimum(total, dl_s) - tpos0, 0, _RPW)
    bound_up = ((bound + _ZN - 1) // _ZN) * _ZN
    table_cp.wait()

    def issue(r, acc):
        srow = jnp.max(plsc.load_gather(idx_v, [jnp.full((_LANES,), r, jnp.int32)]))
        pltpu.make_async_copy(
            table_v.at[pl.ds(srow, 1)],
            out_hbm.at[pl.ds(out_base + r, 1)],
            sem).start()
        return acc

    lax.fori_loop(0, bound_up, issue, 0)

    def issue_zero(g, acc):
        pltpu.make_async_copy(
            table_v.at[pl.ds(_ZROW, _ZN)],
            out_hbm.at[pl.ds(out_base + g * _ZN, _ZN)],
            sem).start()
        return acc

    lax.fori_loop(bound_up // _ZN, _RPW // _ZN, issue_zero, 0)

    def drain(g, acc):
        pltpu.make_async_copy(
            table_v.at[pl.ds(_ZROW, _ZN)],
            out_hbm.at[pl.ds(out_base, _ZN)],
            sem).wait()
        return acc

    lax.fori_loop(0, _RPW // _ZN, drain, 0)


def kernel(doc_s, doc_len, wns):
    wns32 = wns.astype(jnp.int32)
    dl = jnp.broadcast_to(jnp.asarray(doc_len, jnp.int32), (_LANES,))

    run = functools.partial(
        pl.kernel,
        mesh=plsc.VectorSubcoreMesh(core_axis_name="c", subcore_axis_name="s"),
        compiler_params=pltpu.CompilerParams(needs_layout_passes=False),
        out_type=jax.ShapeDtypeStruct((_B * _L, _H), jnp.float32),
        scratch_types=[
            pltpu.VMEM((_S + _ZN, _H), jnp.float32),
            pltpu.VMEM((_S,), jnp.int32),
            pltpu.VMEM((_LANES,), jnp.int32),
            pltpu.VMEM((_S,), jnp.int32),
            pltpu.VMEM((_RPW,), jnp.int32),
            pltpu.SemaphoreType.DMA,
            pltpu.SemaphoreType.DMA,
        ],
    )(_body)

    out = run(doc_s, wns32, dl)
    return out.reshape(_B, _L, _H)


# speedup vs baseline: 20.5793x; 1.0009x over previous
import functools

import jax
import jax.numpy as jnp
from jax import lax
from jax.experimental import pallas as pl
from jax.experimental.pallas import tpu as pltpu
from jax.experimental.pallas import tpu_sc as plsc

_B, _S, _H, _L = 16, 64, 1024, 2048
_LANES = 16
_NW = 32
_RPW = (_B * _L) // _NW
_ZROW = _S
_ZN = 16


def _body(doc_hbm, wns_hbm, dl_hbm, out_hbm,
          table_v, wns_v, dl_v, ends_v, idx_v, sem, tsem):
    nc = 2
    wid = lax.axis_index("s") * nc + lax.axis_index("c")
    b = wid // 2
    half = wid % 2
    out_base = wid * _RPW
    tpos0 = half * _RPW

    pltpu.sync_copy(wns_hbm.at[b], wns_v)
    pltpu.sync_copy(dl_hbm, dl_v)
    table_cp = pltpu.make_async_copy(
        doc_hbm.at[b], table_v.at[pl.ds(0, _S)], tsem)
    table_cp.start()
    fz = jnp.zeros((_LANES,), jnp.float32)

    def zrow(i, acc):
        for j in range(_H // _LANES):
            table_v[_ZROW + i, pl.ds(j * _LANES, _LANES)] = fz
        return acc

    lax.fori_loop(0, _ZN, zrow, 0)

    zcarry = jnp.int32(0)
    carry = jnp.int32(0)
    for c in range(_S // _LANES):
        wv = wns_v[pl.ds(c * _LANES, _LANES)]
        cz = plsc.cumsum((wv == 0).astype(jnp.int32)) + zcarry
        eff = jnp.where(cz == 0, wv, 0)
        ce = plsc.cumsum(eff) + carry
        ends_v[pl.ds(c * _LANES, _LANES)] = ce
        zcarry = jnp.max(cz)
        carry = jnp.max(ce)

    dlv = dl_v[...]
    lane = lax.iota(jnp.int32, _LANES)
    e_last = plsc.load_gather(ends_v, [jnp.full((_LANES,), _S - 1, jnp.int32)])

    def idx_group(i, acc):
        t = tpos0 + i * _LANES + lane
        sid = jnp.zeros((_LANES,), jnp.int32)
        for step in (32, 16, 8, 4, 2, 1):
            probe = sid + (step - 1)
            e = plsc.load_gather(ends_v, [probe])
            sid = jnp.where(e <= t, sid + step, sid)
        valid = (e_last > t) & (t < dlv)
        idx_v[pl.ds(i * _LANES, _LANES)] = jnp.where(valid, sid, _ZROW)
        return acc

    total = carry
    dl_s = jnp.max(dlv)
    bound = jnp.clip(jnp.minimum(total, dl_s) - tpos0, 0, _RPW)
    bound_up = ((bound + _ZN - 1) // _ZN) * _ZN
    lax.fori_loop(0, bound_up // _LANES, idx_group, 0)
    table_cp.wait()

    def issue(r, acc):
        srow = jnp.max(plsc.load_gather(idx_v, [jnp.full((_LANES,), r, jnp.int32)]))
        pltpu.make_async_copy(
            table_v.at[pl.ds(srow, 1)],
            out_hbm.at[pl.ds(out_base + r, 1)],
            sem).start()
        return acc

    lax.fori_loop(0, bound_up, issue, 0)

    def issue_zero(g, acc):
        pltpu.make_async_copy(
            table_v.at[pl.ds(_ZROW, _ZN)],
            out_hbm.at[pl.ds(out_base + g * _ZN, _ZN)],
            sem).start()
        return acc

    lax.fori_loop(bound_up // _ZN, _RPW // _ZN, issue_zero, 0)

    def drain(g, acc):
        pltpu.make_async_copy(
            table_v.at[pl.ds(_ZROW, _ZN)],
            out_hbm.at[pl.ds(out_base, _ZN)],
            sem).wait()
        return acc

    lax.fori_loop(0, _RPW // _ZN, drain, 0)


def kernel(doc_s, doc_len, wns):
    wns32 = wns.astype(jnp.int32)
    dl = jnp.broadcast_to(jnp.asarray(doc_len, jnp.int32), (_LANES,))

    run = functools.partial(
        pl.kernel,
        mesh=plsc.VectorSubcoreMesh(core_axis_name="c", subcore_axis_name="s"),
        compiler_params=pltpu.CompilerParams(needs_layout_passes=False),
        out_type=jax.ShapeDtypeStruct((_B * _L, _H), jnp.float32),
        scratch_types=[
            pltpu.VMEM((_S + _ZN, _H), jnp.float32),
            pltpu.VMEM((_S,), jnp.int32),
            pltpu.VMEM((_LANES,), jnp.int32),
            pltpu.VMEM((_S,), jnp.int32),
            pltpu.VMEM((_RPW,), jnp.int32),
            pltpu.SemaphoreType.DMA,
            pltpu.SemaphoreType.DMA,
        ],
    )(_body)

    out = run(doc_s, wns32, dl)
    return out.reshape(_B, _L, _H)


# speedup vs baseline: 20.5963x; 1.0008x over previous
import functools

import jax
import jax.numpy as jnp
from jax import lax
from jax.experimental import pallas as pl
from jax.experimental.pallas import tpu as pltpu
from jax.experimental.pallas import tpu_sc as plsc

_B, _S, _H, _L = 16, 64, 1024, 2048
_LANES = 16
_NW = 32
_RPW = (_B * _L) // _NW
_ZROW = _S
_ZN = 32


def _body(doc_hbm, wns_hbm, dl_hbm, out_hbm,
          table_v, wns_v, dl_v, ends_v, idx_v, sem, tsem):
    nc = 2
    wid = lax.axis_index("s") * nc + lax.axis_index("c")
    b = wid // 2
    half = wid % 2
    out_base = wid * _RPW
    tpos0 = half * _RPW

    pltpu.sync_copy(wns_hbm.at[b], wns_v)
    pltpu.sync_copy(dl_hbm, dl_v)
    table_cp = pltpu.make_async_copy(
        doc_hbm.at[b], table_v.at[pl.ds(0, _S)], tsem)
    table_cp.start()
    fz = jnp.zeros((_LANES,), jnp.float32)

    def zrow(i, acc):
        for j in range(_H // _LANES):
            table_v[_ZROW + i, pl.ds(j * _LANES, _LANES)] = fz
        return acc

    lax.fori_loop(0, _ZN, zrow, 0)

    zcarry = jnp.int32(0)
    carry = jnp.int32(0)
    for c in range(_S // _LANES):
        wv = wns_v[pl.ds(c * _LANES, _LANES)]
        cz = plsc.cumsum((wv == 0).astype(jnp.int32)) + zcarry
        eff = jnp.where(cz == 0, wv, 0)
        ce = plsc.cumsum(eff) + carry
        ends_v[pl.ds(c * _LANES, _LANES)] = ce
        zcarry = jnp.max(cz)
        carry = jnp.max(ce)

    dlv = dl_v[...]
    lane = lax.iota(jnp.int32, _LANES)
    e_last = plsc.load_gather(ends_v, [jnp.full((_LANES,), _S - 1, jnp.int32)])

    def idx_group(i, acc):
        t = tpos0 + i * _LANES + lane
        sid = jnp.zeros((_LANES,), jnp.int32)
        for step in (32, 16, 8, 4, 2, 1):
            probe = sid + (step - 1)
            e = plsc.load_gather(ends_v, [probe])
            sid = jnp.where(e <= t, sid + step, sid)
        valid = (e_last > t) & (t < dlv)
        idx_v[pl.ds(i * _LANES, _LANES)] = jnp.where(valid, sid, _ZROW)
        return acc

    total = carry
    dl_s = jnp.max(dlv)
    bound = jnp.clip(jnp.minimum(total, dl_s) - tpos0, 0, _RPW)
    bound_up = ((bound + _ZN - 1) // _ZN) * _ZN
    lax.fori_loop(0, bound_up // _LANES, idx_group, 0)
    table_cp.wait()

    def issue(r, acc):
        srow = jnp.max(plsc.load_gather(idx_v, [jnp.full((_LANES,), r, jnp.int32)]))
        pltpu.make_async_copy(
            table_v.at[pl.ds(srow, 1)],
            out_hbm.at[pl.ds(out_base + r, 1)],
            sem).start()
        return acc

    lax.fori_loop(0, bound_up, issue, 0)

    def issue_zero(g, acc):
        pltpu.make_async_copy(
            table_v.at[pl.ds(_ZROW, _ZN)],
            out_hbm.at[pl.ds(out_base + g * _ZN, _ZN)],
            sem).start()
        return acc

    lax.fori_loop(bound_up // _ZN, _RPW // _ZN, issue_zero, 0)

    def drain(g, acc):
        pltpu.make_async_copy(
            table_v.at[pl.ds(_ZROW, _ZN)],
            out_hbm.at[pl.ds(out_base, _ZN)],
            sem).wait()
        return acc

    lax.fori_loop(0, _RPW // _ZN, drain, 0)


def kernel(doc_s, doc_len, wns):
    wns32 = wns.astype(jnp.int32)
    dl = jnp.broadcast_to(jnp.asarray(doc_len, jnp.int32), (_LANES,))

    run = functools.partial(
        pl.kernel,
        mesh=plsc.VectorSubcoreMesh(core_axis_name="c", subcore_axis_name="s"),
        compiler_params=pltpu.CompilerParams(needs_layout_passes=False),
        out_type=jax.ShapeDtypeStruct((_B * _L, _H), jnp.float32),
        scratch_types=[
            pltpu.VMEM((_S + _ZN, _H), jnp.float32),
            pltpu.VMEM((_S,), jnp.int32),
            pltpu.VMEM((_LANES,), jnp.int32),
            pltpu.VMEM((_S,), jnp.int32),
            pltpu.VMEM((_RPW,), jnp.int32),
            pltpu.SemaphoreType.DMA,
            pltpu.SemaphoreType.DMA,
        ],
    )(_body)

    out = run(doc_s, wns32, dl)
    return out.reshape(_B, _L, _H)
